# baseline (speedup 1.0000x reference)
import functools

import numpy as np
import jax
import jax.numpy as jnp
from jax import lax
from jax.experimental import pallas as pl
from jax.experimental.pallas import tpu as pltpu
from jax.experimental.pallas import tpu_sc as plsc

G = 16
P = 1024
S1 = 512
S2 = 128
K = 64
NW = 32
R1SQ = np.float32(0.1 * 0.1)
R2SQ = np.float32(0.2 * 0.2)
NEG = np.float32(-1e30)


def _fps_stage(X, Y, Z, n_out):
    g, p = X.shape
    lane = jax.lax.broadcasted_iota(jnp.int32, (g, p), 1)
    lane_out = jax.lax.broadcasted_iota(jnp.int32, (g, n_out), 1)

    def sel(nxt):
        m = lane == nxt
        xl = jnp.sum(jnp.where(m, X, 0.0), axis=1, keepdims=True)
        yl = jnp.sum(jnp.where(m, Y, 0.0), axis=1, keepdims=True)
        zl = jnp.sum(jnp.where(m, Z, 0.0), axis=1, keepdims=True)
        return xl, yl, zl

    xl, yl, zl = sel(jnp.zeros((g, 1), jnp.int32))
    aX = jnp.where(lane_out == 0, xl, 0.0)
    aY = jnp.where(lane_out == 0, yl, 0.0)
    aZ = jnp.where(lane_out == 0, zl, 0.0)
    dists0 = jnp.full((g, p), 1e30, jnp.float32)

    def step(i, c):
        dists, xl, yl, zl, aX, aY, aZ = c
        d = ((X - xl) ** 2 + (Y - yl) ** 2) + (Z - zl) ** 2
        dists = jnp.minimum(dists, d)
        mx = jnp.max(dists, axis=1, keepdims=True)
        nxt = jnp.min(jnp.where(dists == mx, lane, p), axis=1, keepdims=True)
        xl, yl, zl = sel(nxt)
        aX = jnp.where(lane_out == i, xl, aX)
        aY = jnp.where(lane_out == i, yl, aY)
        aZ = jnp.where(lane_out == i, zl, aZ)
        return (dists, xl, yl, zl, aX, aY, aZ)

    c = jax.lax.fori_loop(1, n_out, step, (dists0, xl, yl, zl, aX, aY, aZ))
    return c[4], c[5], c[6]


def _fps_body(x_ref, y_ref, z_ref,
              x1_ref, y1_ref, z1_ref, x2_ref, y2_ref, z2_ref):
    X = x_ref[...]
    Y = y_ref[...]
    Z = z_ref[...]
    X1, Y1, Z1 = _fps_stage(X, Y, Z, S1)
    x1_ref[...] = X1
    y1_ref[...] = Y1
    z1_ref[...] = Z1
    X2, Y2, Z2 = _fps_stage(X1, Y1, Z1, S2)
    x2_ref[...] = X2
    y2_ref[...] = Y2
    z2_ref[...] = Z2


def _make_compact(cands, ncent, rsq, need_idx, idx_base):
    CT = G * ncent
    ncw = CT // NW
    wpg = ncent // ncw
    ngr = ncw // 16
    nv = cands // 16

    f32 = jnp.float32
    i32 = jnp.int32
    outs = [jax.ShapeDtypeStruct((CT * K,), f32)] * 4
    scratch = [pltpu.VMEM((cands,), f32)] * 3 + \
              [pltpu.VMEM((ncw,), f32)] * 3 + \
              [pltpu.VMEM((ncw * K,), f32)] * 4
    if need_idx:
        outs = outs + [jax.ShapeDtypeStruct((CT * K,), i32)]
        scratch = scratch + [pltpu.VMEM((ncw * K,), i32)]

    mesh = plsc.VectorSubcoreMesh(core_axis_name="c", subcore_axis_name="s")

    @functools.partial(
        pl.kernel, mesh=mesh, out_type=outs, scratch_types=scratch,
        compiler_params=pltpu.CompilerParams(needs_layout_passes=False))
    def body(*refs):
        (xh, yh, zh, cxh, cyh, czh) = refs[:6]
        if need_idx:
            (rxh, ryh, rzh, penh, idxh) = refs[6:11]
            (xv, yv, zv, cxv, cyv, czv, rxv, ryv, rzv, penv, idxv) = refs[11:]
        else:
            (rxh, ryh, rzh, penh) = refs[6:10]
            (xv, yv, zv, cxv, cyv, czv, rxv, ryv, rzv, penv) = refs[10:]

        wid = lax.axis_index("s") * 2 + lax.axis_index("c")
        g = wid // wpg
        off = (wid % wpg) * ncw

        pltpu.sync_copy(xh.at[g], xv)
        pltpu.sync_copy(yh.at[g], yv)
        pltpu.sync_copy(zh.at[g], zv)
        pltpu.sync_copy(cxh.at[g, pl.ds(off, ncw)], cxv)
        pltpu.sync_copy(cyh.at[g, pl.ds(off, ncw)], cyv)
        pltpu.sync_copy(czh.at[g, pl.ds(off, ncw)], czv)

        zero16 = jnp.zeros((16,), f32)
        neg16 = jnp.full((16,), NEG, f32)
        safe16 = jnp.full((16,), g * idx_base, i32)

        def init_slot(t, _):
            sl = pl.ds(t * 16, 16)
            rxv[sl] = zero16
            ryv[sl] = zero16
            rzv[sl] = zero16
            penv[sl] = neg16
            if need_idx:
                idxv[sl] = safe16
            return 0

        lax.fori_loop(0, ncw * K // 16, init_slot, 0)

        lanes = lax.broadcasted_iota(i32, (16,), 0)
        kmax = jnp.full((16,), K - 1, i32)

        def group(grp, _):
            rowbase = (grp * 16 + lanes) * K
            cx = cxv[pl.ds(grp * 16, 16)]
            cy = cyv[pl.ds(grp * 16, 16)]
            cz = czv[pl.ds(grp * 16, 16)]

            def cand(j, cnt):
                jj = jnp.zeros((16,), i32) + j
                xj = plsc.load_gather(xv, [jj])
                yj = plsc.load_gather(yv, [jj])
                zj = plsc.load_gather(zv, [jj])
                dx = xj - cx
                dy = yj - cy
                dz = zj - cz
                d2 = (dx * dx + dy * dy) + dz * dz
                m = d2 <= rsq
                pos = rowbase + jnp.minimum(cnt, kmax)
                plsc.store_scatter(rxv, [pos], dx, mask=m)
                plsc.store_scatter(ryv, [pos], dy, mask=m)
                plsc.store_scatter(rzv, [pos], dz, mask=m)
                plsc.store_scatter(penv, [pos], zero16, mask=m)
                if need_idx:
                    plsc.store_scatter(idxv, [pos], safe16 + j, mask=m)
                return cnt + jnp.where(m, 1, 0).astype(i32)

            lax.fori_loop(0, cands, cand, jnp.zeros((16,), i32))
            return 0

        lax.fori_loop(0, ngr, group, 0)

        base = wid * ncw * K
        pltpu.sync_copy(rxv, rxh.at[pl.ds(base, ncw * K)])
        pltpu.sync_copy(ryv, ryh.at[pl.ds(base, ncw * K)])
        pltpu.sync_copy(rzv, rzh.at[pl.ds(base, ncw * K)])
        pltpu.sync_copy(penv, penh.at[pl.ds(base, ncw * K)])
        if need_idx:
            pltpu.sync_copy(idxv, idxh.at[pl.ds(base, ncw * K)])

    return body


def _gather_rows(xflat, idxflat, nrows, feat, srows):
    per_w = nrows // NW
    CH = 256
    nch = per_w // CH
    f32 = jnp.float32

    mesh = plsc.VectorSubcoreMesh(core_axis_name="c", subcore_axis_name="s")

    @functools.partial(
        pl.kernel, mesh=mesh,
        out_type=jax.ShapeDtypeStruct((nrows * feat,), f32),
        scratch_types=[pltpu.VMEM((srows * feat,), f32),
                       pltpu.VMEM((per_w,), jnp.int32),
                       pltpu.VMEM((CH * feat,), f32)],
        compiler_params=pltpu.CompilerParams(needs_layout_passes=False))
    def body(xh, idxh, oh, xv, idxv, outb):
        wid = lax.axis_index("s") * 2 + lax.axis_index("c")
        g = wid // 2
        pltpu.sync_copy(xh.at[g], xv)
        pltpu.sync_copy(idxh.at[pl.ds(wid * per_w, per_w)], idxv)
        lanes = lax.broadcasted_iota(jnp.int32, (16,), 0)
        ones = jnp.ones((16,), jnp.bool_)

        for ch in range(nch):
            def sg_body(sg, _):
                iv = idxv[pl.ds(ch * CH + sg * 16, 16)]
                wb = iv * feat
                ob = (sg * 16 + lanes) * feat
                for t in range(feat):
                    vals = plsc.load_gather(xv, [wb + t])
                    plsc.store_scatter(outb, [ob + t], vals, mask=ones)
                return 0

            lax.fori_loop(0, CH // 16, sg_body, 0)
            pltpu.sync_copy(
                outb, oh.at[pl.ds((wid * per_w + ch * CH) * feat, CH * feat)])

    return body(xflat, idxflat)


def _conv1_body(rel, pen, w0, b0, w1, b1, w2, b2, out_ref):
    NCB = 256
    h1 = jnp.maximum(rel[...] @ w0[...] + b0[...], 0.0)
    h2 = jnp.maximum(h1 @ w1[...] + b1[...], 0.0)
    h3 = h2 @ w2[...] + b2[...]
    h3 = h3 + pen[...]
    out_ref[...] = jnp.maximum(h3.reshape(NCB, K, 128).max(axis=1), 0.0)


def _conv2_body(xg, rel, pen, wx, wr, b0, w1, b1, w2, b2, out_ref):
    NCB = 128
    h1 = jnp.maximum(xg[...] @ wx[...] + rel[...] @ wr[...] + b0[...], 0.0)
    h2 = jnp.maximum(h1 @ w1[...] + b1[...], 0.0)
    h3 = h2 @ w2[...] + b2[...]
    h3 = h3 + pen[...]
    out_ref[...] = jnp.maximum(h3.reshape(NCB, K, 256).max(axis=1), 0.0)


def _final_body(x2_ref, cpos2m, wga, wgb, bg0, wg1, bg1, wg2, bg2,
                wl1, bl1, wl2, bl2, wl3, bl3, out_ref):
    h = jnp.maximum(x2_ref[0] @ wga[...] + cpos2m[0] @ wgb[...] + bg0[...], 0.0)
    h = jnp.maximum(h @ wg1[...] + bg1[...], 0.0)
    h = h @ wg2[...] + bg2[...]
    f = jnp.max(h, axis=0, keepdims=True)
    h = jnp.maximum(f @ wl1[...] + bl1[...], 0.0)
    h = jnp.maximum(h @ wl2[...] + bl2[...], 0.0)
    lg = h @ wl3[...] + bl3[...]
    m = jnp.max(lg, axis=1, keepdims=True)
    s = lg - m
    lse = jnp.log(jnp.sum(jnp.exp(s), axis=1, keepdims=True))
    out_ref[0] = s - lse


def _full(shape):
    return pl.BlockSpec(shape, lambda *_: tuple(0 for _ in shape))


def kernel(pos, batch, w1_0, b1_0, w1_1, b1_1, w1_2, b1_2,
           w2_0, b2_0, w2_1, b2_1, w2_2, b2_2,
           wg_0, bg_0, wg_1, bg_1, wg_2, bg_2,
           wl1, bl1, wl2, bl2, wl3, bl3):
    posb = pos.reshape(G, P, 3)
    X = posb[:, :, 0]
    Y = posb[:, :, 1]
    Z = posb[:, :, 2]
    f32 = jnp.float32

    fps = pl.pallas_call(
        _fps_body,
        out_shape=[jax.ShapeDtypeStruct((G, S1), f32)] * 3
        + [jax.ShapeDtypeStruct((G, S2), f32)] * 3,
    )
    X1, Y1, Z1, X2, Y2, Z2 = fps(X, Y, Z)

    compact1 = _make_compact(P, S1, R1SQ, False, S1)
    rx1, ry1, rz1, pen1 = compact1(X, Y, Z, X1, Y1, Z1)
    rel1 = jnp.stack([rx1, ry1, rz1], axis=-1)
    pen1 = pen1.reshape(G * S1 * K, 1)

    TILES1 = 32
    NCB1 = G * S1 // TILES1
    conv1 = pl.pallas_call(
        _conv1_body,
        grid=(TILES1,),
        in_specs=[pl.BlockSpec((NCB1 * K, 3), lambda t: (t, 0)),
                  pl.BlockSpec((NCB1 * K, 1), lambda t: (t, 0)),
                  _full(w1_0.shape), _full((1, 64)),
                  _full(w1_1.shape), _full((1, 64)),
                  _full(w1_2.shape), _full((1, 128))],
        out_specs=pl.BlockSpec((NCB1, 128), lambda t: (t, 0)),
        out_shape=jax.ShapeDtypeStruct((G * S1, 128), f32),
    )
    x = conv1(rel1, pen1,
              w1_0, b1_0.reshape(1, -1), w1_1, b1_1.reshape(1, -1),
              w1_2, b1_2.reshape(1, -1))

    compact2 = _make_compact(S1, S2, R2SQ, True, 0)
    rx2, ry2, rz2, pen2, idx2 = compact2(X1, Y1, Z1, X2, Y2, Z2)
    rel2 = jnp.stack([rx2, ry2, rz2], axis=-1)
    pen2 = pen2.reshape(G * S2 * K, 1)

    xg = _gather_rows(x.reshape(G, S1 * 128), idx2,
                      G * S2 * K, 128, S1).reshape(G * S2 * K, 128)

    TILES2 = 16
    NCB2 = G * S2 // TILES2
    conv2 = pl.pallas_call(
        _conv2_body,
        grid=(TILES2,),
        in_specs=[pl.BlockSpec((NCB2 * K, 128), lambda t: (t, 0)),
                  pl.BlockSpec((NCB2 * K, 3), lambda t: (t, 0)),
                  pl.BlockSpec((NCB2 * K, 1), lambda t: (t, 0)),
                  _full((128, 128)), _full((3, 128)), _full((1, 128)),
                  _full(w2_1.shape), _full((1, 128)),
                  _full(w2_2.shape), _full((1, 256))],
        out_specs=pl.BlockSpec((NCB2, 256), lambda t: (t, 0)),
        out_shape=jax.ShapeDtypeStruct((G * S2, 256), f32),
    )
    x2 = conv2(xg, rel2, pen2,
               w2_0[:128], w2_0[128:], b2_0.reshape(1, -1),
               w2_1, b2_1.reshape(1, -1), w2_2, b2_2.reshape(1, -1))

    pos2m = jnp.stack([X2, Y2, Z2], axis=-1)
    final = pl.pallas_call(
        _final_body,
        grid=(G,),
        in_specs=[pl.BlockSpec((1, S2, 256), lambda g: (g, 0, 0)),
                  pl.BlockSpec((1, S2, 3), lambda g: (g, 0, 0)),
                  _full((256, 256)), _full((3, 256)), _full((1, 256)),
                  _full(wg_1.shape), _full((1, 512)),
                  _full(wg_2.shape), _full((1, 1024)),
                  _full(wl1.shape), _full((1, 512)),
                  _full(wl2.shape), _full((1, 256)),
                  _full(wl3.shape), _full((1, 10))],
        out_specs=pl.BlockSpec((1, 1, 10), lambda g: (g, 0, 0)),
        out_shape=jax.ShapeDtypeStruct((G, 1, 10), f32),
    )
    out = final(x2.reshape(G, S2, 256), pos2m,
                wg_0[:256], wg_0[256:], bg_0.reshape(1, -1),
                wg_1, bg_1.reshape(1, -1), wg_2, bg_2.reshape(1, -1),
                wl1, bl1.reshape(1, -1), wl2, bl2.reshape(1, -1),
                wl3, bl3.reshape(1, -1))
    return out.reshape(G, 10)


# speedup vs baseline: 1.2660x; 1.2660x over previous
import functools

import numpy as np
import jax
import jax.numpy as jnp
from jax import lax
from jax.experimental import pallas as pl
from jax.experimental.pallas import tpu as pltpu
from jax.experimental.pallas import tpu_sc as plsc

G = 16
P = 1024
S1 = 512
S2 = 128
K = 64
NW = 32
R1SQ = np.float32(0.1 * 0.1)
R2SQ = np.float32(0.2 * 0.2)
NEG = np.float32(-1e30)


def _fps_stage(X, Y, Z, n_out):
    g, p = X.shape
    lane = jax.lax.broadcasted_iota(jnp.int32, (g, p), 1)
    lane_out = jax.lax.broadcasted_iota(jnp.int32, (g, n_out), 1)

    def sel(nxt):
        m = lane == nxt
        xl = jnp.sum(jnp.where(m, X, 0.0), axis=1, keepdims=True)
        yl = jnp.sum(jnp.where(m, Y, 0.0), axis=1, keepdims=True)
        zl = jnp.sum(jnp.where(m, Z, 0.0), axis=1, keepdims=True)
        return xl, yl, zl

    xl, yl, zl = sel(jnp.zeros((g, 1), jnp.int32))
    aX = jnp.where(lane_out == 0, xl, 0.0)
    aY = jnp.where(lane_out == 0, yl, 0.0)
    aZ = jnp.where(lane_out == 0, zl, 0.0)
    dists0 = jnp.full((g, p), 1e30, jnp.float32)

    def step(i, c):
        dists, xl, yl, zl, aX, aY, aZ = c
        d = ((X - xl) ** 2 + (Y - yl) ** 2) + (Z - zl) ** 2
        dists = jnp.minimum(dists, d)
        mx = jnp.max(dists, axis=1, keepdims=True)
        nxt = jnp.min(jnp.where(dists == mx, lane, p), axis=1, keepdims=True)
        xl, yl, zl = sel(nxt)
        aX = jnp.where(lane_out == i, xl, aX)
        aY = jnp.where(lane_out == i, yl, aY)
        aZ = jnp.where(lane_out == i, zl, aZ)
        return (dists, xl, yl, zl, aX, aY, aZ)

    c = jax.lax.fori_loop(1, n_out, step, (dists0, xl, yl, zl, aX, aY, aZ))
    return c[4], c[5], c[6]


def _fps_body(x_ref, y_ref, z_ref,
              x1_ref, y1_ref, z1_ref, x2_ref, y2_ref, z2_ref):
    X = x_ref[...]
    Y = y_ref[...]
    Z = z_ref[...]
    X1, Y1, Z1 = _fps_stage(X, Y, Z, S1)
    x1_ref[...] = X1
    y1_ref[...] = Y1
    z1_ref[...] = Z1
    X2, Y2, Z2 = _fps_stage(X1, Y1, Z1, S2)
    x2_ref[...] = X2
    y2_ref[...] = Y2
    z2_ref[...] = Z2


def _make_compact(cands, ncent, rsq, need_idx, idx_base):
    CT = G * ncent
    ncw = CT // NW
    wpg = ncent // ncw
    ngr = ncw // 16
    nv = cands // 16

    f32 = jnp.float32
    i32 = jnp.int32
    outs = [jax.ShapeDtypeStruct((CT * K,), f32)] * 4
    scratch = [pltpu.VMEM((cands,), f32)] * 3 + \
              [pltpu.VMEM((ncw,), f32)] * 3 + \
              [pltpu.VMEM((ncw * K,), f32)] * 4
    if need_idx:
        outs = outs + [jax.ShapeDtypeStruct((CT * K,), i32)]
        scratch = scratch + [pltpu.VMEM((ncw * K,), i32)]

    mesh = plsc.VectorSubcoreMesh(core_axis_name="c", subcore_axis_name="s")

    @functools.partial(
        pl.kernel, mesh=mesh, out_type=outs, scratch_types=scratch,
        compiler_params=pltpu.CompilerParams(needs_layout_passes=False))
    def body(*refs):
        (xh, yh, zh, cxh, cyh, czh) = refs[:6]
        if need_idx:
            (rxh, ryh, rzh, penh, idxh) = refs[6:11]
            (xv, yv, zv, cxv, cyv, czv, rxv, ryv, rzv, penv, idxv) = refs[11:]
        else:
            (rxh, ryh, rzh, penh) = refs[6:10]
            (xv, yv, zv, cxv, cyv, czv, rxv, ryv, rzv, penv) = refs[10:]

        wid = lax.axis_index("s") * 2 + lax.axis_index("c")
        g = wid // wpg
        off = (wid % wpg) * ncw

        pltpu.sync_copy(xh.at[g], xv)
        pltpu.sync_copy(yh.at[g], yv)
        pltpu.sync_copy(zh.at[g], zv)
        pltpu.sync_copy(cxh.at[g, pl.ds(off, ncw)], cxv)
        pltpu.sync_copy(cyh.at[g, pl.ds(off, ncw)], cyv)
        pltpu.sync_copy(czh.at[g, pl.ds(off, ncw)], czv)

        zero16 = jnp.zeros((16,), f32)
        neg16 = jnp.full((16,), NEG, f32)
        safe16 = jnp.full((16,), g * idx_base, i32)

        def init_slot(t, _):
            sl = pl.ds(t * 16, 16)
            rxv[sl] = zero16
            ryv[sl] = zero16
            rzv[sl] = zero16
            penv[sl] = neg16
            if need_idx:
                idxv[sl] = safe16
            return 0

        lax.fori_loop(0, ncw * K // 16, init_slot, 0)

        lanes = lax.broadcasted_iota(i32, (16,), 0)
        kmax = jnp.full((16,), K - 1, i32)

        def group(grp, _):
            rowbase = (grp * 16 + lanes) * K
            cx = cxv[pl.ds(grp * 16, 16)]
            cy = cyv[pl.ds(grp * 16, 16)]
            cz = czv[pl.ds(grp * 16, 16)]

            def cand(j, cnt):
                jj = jnp.zeros((16,), i32) + j
                xj = plsc.load_gather(xv, [jj])
                yj = plsc.load_gather(yv, [jj])
                zj = plsc.load_gather(zv, [jj])
                dx = xj - cx
                dy = yj - cy
                dz = zj - cz
                d2 = (dx * dx + dy * dy) + dz * dz
                m = d2 <= rsq
                pos = rowbase + jnp.minimum(cnt, kmax)
                plsc.store_scatter(rxv, [pos], dx, mask=m)
                plsc.store_scatter(ryv, [pos], dy, mask=m)
                plsc.store_scatter(rzv, [pos], dz, mask=m)
                plsc.store_scatter(penv, [pos], zero16, mask=m)
                if need_idx:
                    plsc.store_scatter(idxv, [pos], safe16 + j, mask=m)
                return cnt + jnp.where(m, 1, 0).astype(i32)

            lax.fori_loop(0, cands, cand, jnp.zeros((16,), i32))
            return 0

        lax.fori_loop(0, ngr, group, 0)

        base = wid * ncw * K
        pltpu.sync_copy(rxv, rxh.at[pl.ds(base, ncw * K)])
        pltpu.sync_copy(ryv, ryh.at[pl.ds(base, ncw * K)])
        pltpu.sync_copy(rzv, rzh.at[pl.ds(base, ncw * K)])
        pltpu.sync_copy(penv, penh.at[pl.ds(base, ncw * K)])
        if need_idx:
            pltpu.sync_copy(idxv, idxh.at[pl.ds(base, ncw * K)])

    return body


def _gather_rows(xflat, idxflat, nrows, feat, srows):
    per_w = nrows // NW
    CH = 256
    nch = per_w // CH
    f32 = jnp.float32

    mesh = plsc.VectorSubcoreMesh(core_axis_name="c", subcore_axis_name="s")

    @functools.partial(
        pl.kernel, mesh=mesh,
        out_type=jax.ShapeDtypeStruct((nrows * feat,), f32),
        scratch_types=[pltpu.VMEM((srows * feat,), f32),
                       pltpu.VMEM((per_w,), jnp.int32),
                       pltpu.VMEM((CH * feat,), f32)],
        compiler_params=pltpu.CompilerParams(needs_layout_passes=False))
    def body(xh, idxh, oh, xv, idxv, outb):
        wid = lax.axis_index("s") * 2 + lax.axis_index("c")
        g = wid // 2
        pltpu.sync_copy(xh.at[g], xv)
        pltpu.sync_copy(idxh.at[pl.ds(wid * per_w, per_w)], idxv)
        lanes = lax.broadcasted_iota(jnp.int32, (16,), 0)
        ones = jnp.ones((16,), jnp.bool_)

        for ch in range(nch):
            def sg_body(sg, _):
                iv = idxv[pl.ds(ch * CH + sg * 16, 16)]
                wb = iv * feat
                ob = (sg * 16 + lanes) * feat
                for t in range(feat):
                    vals = plsc.load_gather(xv, [wb + t])
                    plsc.store_scatter(outb, [ob + t], vals, mask=ones)
                return 0

            lax.fori_loop(0, CH // 16, sg_body, 0)
            pltpu.sync_copy(
                outb, oh.at[pl.ds((wid * per_w + ch * CH) * feat, CH * feat)])

    return body(xflat, idxflat)


def _conv1_body(rel, pen, w0, b0, w1, b1, w2, b2, out_ref):
    NCB = 256
    h1 = jnp.maximum(rel[...] @ w0[...] + b0[...], 0.0)
    h2 = jnp.maximum(h1 @ w1[...] + b1[...], 0.0)
    h3 = h2 @ w2[...] + b2[...]
    h3 = h3 + pen[...]
    out_ref[...] = jnp.maximum(h3.reshape(NCB, K, 128).max(axis=1), 0.0)


def _conv2_body(x_in, idxc, rel, pen, wx, wr, b0, w1, b1, w2, b2, out_ref):
    NCB = 128
    jio = jax.lax.broadcasted_iota(jnp.int32, (NCB * K, S1), 1)
    ohf = jnp.where(idxc[...] == jio, 1.0, 0.0)
    xg = ohf @ x_in[...]
    h1 = jnp.maximum(xg @ wx[...] + rel[...] @ wr[...] + b0[...], 0.0)
    h2 = jnp.maximum(h1 @ w1[...] + b1[...], 0.0)
    h3 = h2 @ w2[...] + b2[...]
    h3 = h3 + pen[...]
    out_ref[...] = jnp.maximum(h3.reshape(NCB, K, 256).max(axis=1), 0.0)


def _final_body(x2_ref, cpos2m, wga, wgb, bg0, wg1, bg1, wg2, bg2,
                wl1, bl1, wl2, bl2, wl3, bl3, out_ref):
    h = jnp.maximum(x2_ref[0] @ wga[...] + cpos2m[0] @ wgb[...] + bg0[...], 0.0)
    h = jnp.maximum(h @ wg1[...] + bg1[...], 0.0)
    h = h @ wg2[...] + bg2[...]
    f = jnp.max(h, axis=0, keepdims=True)
    h = jnp.maximum(f @ wl1[...] + bl1[...], 0.0)
    h = jnp.maximum(h @ wl2[...] + bl2[...], 0.0)
    lg = h @ wl3[...] + bl3[...]
    m = jnp.max(lg, axis=1, keepdims=True)
    s = lg - m
    lse = jnp.log(jnp.sum(jnp.exp(s), axis=1, keepdims=True))
    out_ref[0] = s - lse


def _full(shape):
    return pl.BlockSpec(shape, lambda *_: tuple(0 for _ in shape))


def kernel(pos, batch, w1_0, b1_0, w1_1, b1_1, w1_2, b1_2,
           w2_0, b2_0, w2_1, b2_1, w2_2, b2_2,
           wg_0, bg_0, wg_1, bg_1, wg_2, bg_2,
           wl1, bl1, wl2, bl2, wl3, bl3):
    posb = pos.reshape(G, P, 3)
    X = posb[:, :, 0]
    Y = posb[:, :, 1]
    Z = posb[:, :, 2]
    f32 = jnp.float32

    fps = pl.pallas_call(
        _fps_body,
        out_shape=[jax.ShapeDtypeStruct((G, S1), f32)] * 3
        + [jax.ShapeDtypeStruct((G, S2), f32)] * 3,
    )
    X1, Y1, Z1, X2, Y2, Z2 = fps(X, Y, Z)

    compact1 = _make_compact(P, S1, R1SQ, False, S1)
    rx1, ry1, rz1, pen1 = compact1(X, Y, Z, X1, Y1, Z1)
    rel1 = jnp.stack([rx1, ry1, rz1], axis=-1)
    pen1 = pen1.reshape(G * S1 * K, 1)

    TILES1 = 32
    NCB1 = G * S1 // TILES1
    conv1 = pl.pallas_call(
        _conv1_body,
        grid=(TILES1,),
        in_specs=[pl.BlockSpec((NCB1 * K, 3), lambda t: (t, 0)),
                  pl.BlockSpec((NCB1 * K, 1), lambda t: (t, 0)),
                  _full(w1_0.shape), _full((1, 64)),
                  _full(w1_1.shape), _full((1, 64)),
                  _full(w1_2.shape), _full((1, 128))],
        out_specs=pl.BlockSpec((NCB1, 128), lambda t: (t, 0)),
        out_shape=jax.ShapeDtypeStruct((G * S1, 128), f32),
    )
    x = conv1(rel1, pen1,
              w1_0, b1_0.reshape(1, -1), w1_1, b1_1.reshape(1, -1),
              w1_2, b1_2.reshape(1, -1))

    compact2 = _make_compact(S1, S2, R2SQ, True, 0)
    rx2, ry2, rz2, pen2, idx2 = compact2(X1, Y1, Z1, X2, Y2, Z2)
    rel2 = jnp.stack([rx2, ry2, rz2], axis=-1)
    pen2 = pen2.reshape(G * S2 * K, 1)

    TILES2 = 16
    NCB2 = G * S2 // TILES2
    idxc = idx2.reshape(G * S2 * K, 1)
    conv2 = pl.pallas_call(
        _conv2_body,
        grid=(TILES2,),
        in_specs=[pl.BlockSpec((S1, 128), lambda t: (t, 0)),
                  pl.BlockSpec((NCB2 * K, 1), lambda t: (t, 0)),
                  pl.BlockSpec((NCB2 * K, 3), lambda t: (t, 0)),
                  pl.BlockSpec((NCB2 * K, 1), lambda t: (t, 0)),
                  _full((128, 128)), _full((3, 128)), _full((1, 128)),
                  _full(w2_1.shape), _full((1, 128)),
                  _full(w2_2.shape), _full((1, 256))],
        out_specs=pl.BlockSpec((NCB2, 256), lambda t: (t, 0)),
        out_shape=jax.ShapeDtypeStruct((G * S2, 256), f32),
    )
    x2 = conv2(x, idxc, rel2, pen2,
               w2_0[:128], w2_0[128:], b2_0.reshape(1, -1),
               w2_1, b2_1.reshape(1, -1), w2_2, b2_2.reshape(1, -1))

    pos2m = jnp.stack([X2, Y2, Z2], axis=-1)
    final = pl.pallas_call(
        _final_body,
        grid=(G,),
        in_specs=[pl.BlockSpec((1, S2, 256), lambda g: (g, 0, 0)),
                  pl.BlockSpec((1, S2, 3), lambda g: (g, 0, 0)),
                  _full((256, 256)), _full((3, 256)), _full((1, 256)),
                  _full(wg_1.shape), _full((1, 512)),
                  _full(wg_2.shape), _full((1, 1024)),
                  _full(wl1.shape), _full((1, 512)),
                  _full(wl2.shape), _full((1, 256)),
                  _full(wl3.shape), _full((1, 10))],
        out_specs=pl.BlockSpec((1, 1, 10), lambda g: (g, 0, 0)),
        out_shape=jax.ShapeDtypeStruct((G, 1, 10), f32),
    )
    out = final(x2.reshape(G, S2, 256), pos2m,
                wg_0[:256], wg_0[256:], bg_0.reshape(1, -1),
                wg_1, bg_1.reshape(1, -1), wg_2, bg_2.reshape(1, -1),
                wl1, bl1.reshape(1, -1), wl2, bl2.reshape(1, -1),
                wl3, bl3.reshape(1, -1))
    return out.reshape(G, 10)
